# baseline (speedup 1.0000x reference)
import functools

import jax
import jax.numpy as jnp
from jax import lax
from jax.experimental import pallas as pl
from jax.experimental.pallas import tpu as pltpu
from jax.experimental.pallas import tpu_sc as plsc

B = 512
N = 77
NP = 80
IN_FEATS = 256
HIDDEN = 512
OUT_FEATS = 256
E = 2464
ECHUNKS = E // 16

BB = 128

_SC_MESH = plsc.VectorSubcoreMesh(core_axis_name="c", subcore_axis_name="s")


@functools.partial(
    pl.kernel,
    mesh=_SC_MESH,
    out_type=jax.ShapeDtypeStruct((NP * NP,), jnp.float32),
    scratch_types=[
        pltpu.VMEM((2 * E,), jnp.int32),
        pltpu.VMEM((E,), jnp.int32),
        pltpu.VMEM((E,), jnp.float32),
        pltpu.VMEM_SHARED((NP * NP,), jnp.float32),
    ],
)
def _sc_count_kernel(ei_hbm, zeros_hbm, ones_hbm, a_hbm,
                     ei_v, flat_v, ones_v, shared):
    wid = lax.axis_index("s") * 2 + lax.axis_index("c")

    @pl.when(wid == 0)
    def _():
        pltpu.sync_copy(ei_hbm, ei_v)
        pltpu.sync_copy(zeros_hbm, shared)
        pltpu.sync_copy(ones_hbm, ones_v)

        def body(k, carry):
            src = ei_v[pl.ds(k * 16, 16)]
            dst = ei_v[pl.ds(E + k * 16, 16)]
            flat_v[pl.ds(k * 16, 16)] = dst * NP + src
            return carry

        lax.fori_loop(0, ECHUNKS, body, 0)
        pltpu.sync_copy(ones_v, shared.at[flat_v], add=True)
        pltpu.sync_copy(shared, a_hbm)


def _mm(a, b):
    return jax.lax.dot_general(
        a, b, (((1,), (0,)), ((), ())), preferred_element_type=jnp.float32
    )


def _bmm_ahat(ahat_b, v):
    return jax.lax.dot_general(
        ahat_b, v, (((2,), (1,)), ((0,), (0,))),
        preferred_element_type=jnp.float32,
    )


def _gcn_kernel(araw_ref, x_ref, w1_ref, b1_ref, w2_ref, b2_ref, out_ref,
                ahat_scr):
    @pl.when(pl.program_id(0) == 0)
    def _():
        a = araw_ref[...]
        deg_in = jnp.sum(a, axis=1, keepdims=True)
        deg_out = jnp.sum(a, axis=0, keepdims=True)
        norm_dst = jax.lax.rsqrt(jnp.maximum(deg_in, 1.0))
        norm_src = jax.lax.rsqrt(jnp.maximum(deg_out, 1.0))
        ahat_scr[...] = (a * norm_dst * norm_src).astype(jnp.bfloat16)

    ahat_b = jnp.broadcast_to(ahat_scr[...][None], (BB, NP, NP))
    x = x_ref[...].astype(jnp.bfloat16)
    xp = jnp.concatenate(
        [x, jnp.zeros((BB, NP - N, IN_FEATS), jnp.bfloat16)], axis=1)
    y = _bmm_ahat(ahat_b, xp).astype(jnp.bfloat16).reshape(BB * NP, IN_FEATS)
    h = jnp.maximum(_mm(y, w1_ref[:, :]) + b1_ref[:, :],
                    0.0).astype(jnp.bfloat16)
    t = _mm(h, w2_ref[:, :]).astype(jnp.bfloat16).reshape(BB, NP, OUT_FEATS)
    o = _bmm_ahat(ahat_b, t) + b2_ref[:, :][None]
    out_ref[...] = o[:, :N, :].astype(jnp.bfloat16)


def kernel(in_feat, edge_index, W1, b1, W2, b2):
    a_raw = _sc_count_kernel(
        edge_index.reshape(2 * E), jnp.zeros((NP * NP,), jnp.float32),
        jnp.ones((E,), jnp.float32)
    ).reshape(NP, NP)

    grid = (B // BB,)
    out = pl.pallas_call(
        _gcn_kernel,
        grid=grid,
        in_specs=[
            pl.BlockSpec((NP, NP), lambda i: (0, 0)),
            pl.BlockSpec((BB, N, IN_FEATS), lambda i: (i, 0, 0)),
            pl.BlockSpec((IN_FEATS, HIDDEN), lambda i: (0, 0)),
            pl.BlockSpec((1, HIDDEN), lambda i: (0, 0)),
            pl.BlockSpec((HIDDEN, OUT_FEATS), lambda i: (0, 0)),
            pl.BlockSpec((1, OUT_FEATS), lambda i: (0, 0)),
        ],
        out_specs=pl.BlockSpec((BB, N, OUT_FEATS), lambda i: (i, 0, 0)),
        out_shape=jax.ShapeDtypeStruct((B, N, OUT_FEATS), jnp.bfloat16),
        scratch_shapes=[pltpu.VMEM((NP, NP), jnp.bfloat16)],
        compiler_params=pltpu.CompilerParams(
            dimension_semantics=("arbitrary",),
        ),
    )(a_raw, in_feat, W1.astype(jnp.bfloat16),
      b1.astype(jnp.bfloat16).reshape(1, HIDDEN), W2.astype(jnp.bfloat16),
      b2.astype(jnp.bfloat16).reshape(1, OUT_FEATS))
    return out.astype(jnp.float32)


# speedup vs baseline: 1.0070x; 1.0070x over previous
import functools

import jax
import jax.numpy as jnp
from jax import lax
from jax.experimental import pallas as pl
from jax.experimental.pallas import tpu as pltpu
from jax.experimental.pallas import tpu_sc as plsc

B = 512
N = 77
NP = 80
IN_FEATS = 256
HIDDEN = 512
OUT_FEATS = 256
E = 2464
ECHUNKS = E // 16

BB = 64

_SC_MESH = plsc.VectorSubcoreMesh(core_axis_name="c", subcore_axis_name="s")


@functools.partial(
    pl.kernel,
    mesh=_SC_MESH,
    out_type=jax.ShapeDtypeStruct((NP * NP,), jnp.float32),
    scratch_types=[
        pltpu.VMEM((2 * E,), jnp.int32),
        pltpu.VMEM((E,), jnp.int32),
        pltpu.VMEM((E,), jnp.float32),
        pltpu.VMEM_SHARED((NP * NP,), jnp.float32),
    ],
)
def _sc_count_kernel(ei_hbm, zeros_hbm, ones_hbm, a_hbm,
                     ei_v, flat_v, ones_v, shared):
    wid = lax.axis_index("s") * 2 + lax.axis_index("c")

    @pl.when(wid == 0)
    def _():
        pltpu.sync_copy(ei_hbm, ei_v)
        pltpu.sync_copy(zeros_hbm, shared)
        pltpu.sync_copy(ones_hbm, ones_v)

        def body(k, carry):
            src = ei_v[pl.ds(k * 16, 16)]
            dst = ei_v[pl.ds(E + k * 16, 16)]
            flat_v[pl.ds(k * 16, 16)] = dst * NP + src
            return carry

        lax.fori_loop(0, ECHUNKS, body, 0)
        pltpu.sync_copy(ones_v, shared.at[flat_v], add=True)
        pltpu.sync_copy(shared, a_hbm)


def _mm(a, b):
    return jax.lax.dot_general(
        a, b, (((1,), (0,)), ((), ())), preferred_element_type=jnp.float32
    )


def _bmm_ahat(ahat_b, v):
    return jax.lax.dot_general(
        ahat_b, v, (((2,), (1,)), ((0,), (0,))),
        preferred_element_type=jnp.float32,
    )


def _gcn_kernel(araw_ref, x_ref, w1_ref, b1_ref, w2_ref, b2_ref, out_ref,
                ahat_scr):
    @pl.when(pl.program_id(0) == 0)
    def _():
        a = araw_ref[...]
        deg_in = jnp.sum(a, axis=1, keepdims=True)
        deg_out = jnp.sum(a, axis=0, keepdims=True)
        norm_dst = jax.lax.rsqrt(jnp.maximum(deg_in, 1.0))
        norm_src = jax.lax.rsqrt(jnp.maximum(deg_out, 1.0))
        ahat_scr[...] = (a * norm_dst * norm_src).astype(jnp.bfloat16)

    ahat_b = jnp.broadcast_to(ahat_scr[...][None], (BB, NP, NP))
    x = x_ref[...].astype(jnp.bfloat16)
    xp = jnp.concatenate(
        [x, jnp.zeros((BB, NP - N, IN_FEATS), jnp.bfloat16)], axis=1)
    y = _bmm_ahat(ahat_b, xp).astype(jnp.bfloat16).reshape(BB * NP, IN_FEATS)
    h = jnp.maximum(_mm(y, w1_ref[:, :]) + b1_ref[:, :],
                    0.0).astype(jnp.bfloat16)
    t = _mm(h, w2_ref[:, :]).astype(jnp.bfloat16).reshape(BB, NP, OUT_FEATS)
    o = _bmm_ahat(ahat_b, t) + b2_ref[:, :][None]
    out_ref[...] = o[:, :N, :].astype(jnp.bfloat16)


def kernel(in_feat, edge_index, W1, b1, W2, b2):
    a_raw = _sc_count_kernel(
        edge_index.reshape(2 * E), jnp.zeros((NP * NP,), jnp.float32),
        jnp.ones((E,), jnp.float32)
    ).reshape(NP, NP)

    grid = (B // BB,)
    out = pl.pallas_call(
        _gcn_kernel,
        grid=grid,
        in_specs=[
            pl.BlockSpec((NP, NP), lambda i: (0, 0)),
            pl.BlockSpec((BB, N, IN_FEATS), lambda i: (i, 0, 0)),
            pl.BlockSpec((IN_FEATS, HIDDEN), lambda i: (0, 0)),
            pl.BlockSpec((1, HIDDEN), lambda i: (0, 0)),
            pl.BlockSpec((HIDDEN, OUT_FEATS), lambda i: (0, 0)),
            pl.BlockSpec((1, OUT_FEATS), lambda i: (0, 0)),
        ],
        out_specs=pl.BlockSpec((BB, N, OUT_FEATS), lambda i: (i, 0, 0)),
        out_shape=jax.ShapeDtypeStruct((B, N, OUT_FEATS), jnp.bfloat16),
        scratch_shapes=[pltpu.VMEM((NP, NP), jnp.bfloat16)],
        compiler_params=pltpu.CompilerParams(
            dimension_semantics=("arbitrary",),
        ),
    )(a_raw, in_feat, W1.astype(jnp.bfloat16),
      b1.astype(jnp.bfloat16).reshape(1, HIDDEN), W2.astype(jnp.bfloat16),
      b2.astype(jnp.bfloat16).reshape(1, OUT_FEATS))
    return out.astype(jnp.float32)
